# baseline (speedup 1.0000x reference)
import functools

import jax
import jax.numpy as jnp
from jax.experimental import pallas as pl
from jax.experimental.pallas import tpu as pltpu

N_TOK = 4096
D = 32
K = 8192
BN = 512


def _norm_body(feat_ref, cb_ref, fn_ref, cn_ref):
    f = feat_ref[...]
    fn_ref[...] = f / (jnp.sqrt(jnp.sum(f * f, axis=1, keepdims=True)) + 1e-12)
    c = cb_ref[...]
    cn_ref[...] = c / (jnp.sqrt(jnp.sum(c * c, axis=1, keepdims=True)) + 1e-12)


def _argmin_body(fn_ref, cn_ref, ki_ref, idx_ref, s_ref):
    n = pl.program_id(0)
    nb = pl.num_programs(0) - 1

    @pl.when(n < nb)
    def _mm():
        s_ref[n % 2] = jax.lax.dot_general(
            fn_ref[...], cn_ref[...],
            dimension_numbers=(((1,), (1,)), ((), ())),
            preferred_element_type=jnp.float32)

    @pl.when(n > 0)
    def _tail():
        s = s_ref[(n - 1) % 2]
        d = 1.0 - s
        dmin = jnp.min(d, axis=1, keepdims=True)
        midx = jnp.min(
            jnp.where(d == dmin, ki_ref[...], jnp.int32(2**31 - 1)),
            axis=1, keepdims=True)
        idx_ref[...] = midx


def _nearest_idx(z, codebook):
    feat = z.reshape(N_TOK, D)
    fn, cn = pl.pallas_call(
        _norm_body,
        grid=(1,),
        in_specs=[
            pl.BlockSpec((N_TOK, D), lambda i: (0, 0)),
            pl.BlockSpec((K, D), lambda i: (0, 0)),
        ],
        out_specs=[
            pl.BlockSpec((N_TOK, D), lambda i: (0, 0)),
            pl.BlockSpec((K, D), lambda i: (0, 0)),
        ],
        out_shape=[
            jax.ShapeDtypeStruct((N_TOK, D), jnp.float32),
            jax.ShapeDtypeStruct((K, D), jnp.float32),
        ],
    )(feat, codebook)
    ki = jax.lax.broadcasted_iota(jnp.int32, (1, K), 1)
    nb = N_TOK // BN
    idx2 = pl.pallas_call(
        _argmin_body,
        grid=(nb + 1,),
        in_specs=[
            pl.BlockSpec((BN, D), lambda n: (jnp.minimum(n, nb - 1), 0)),
            pl.BlockSpec((K, D), lambda n: (0, 0)),
            pl.BlockSpec((1, K), lambda n: (0, 0)),
        ],
        out_specs=pl.BlockSpec((BN, 1), lambda n: (jnp.maximum(n - 1, 0), 0)),
        out_shape=jax.ShapeDtypeStruct((N_TOK, 1), jnp.int32),
        scratch_shapes=[pltpu.VMEM((2, BN, K), jnp.float32)],
    )(fn, cn, ki)
    return idx2.reshape(N_TOK)


def kernel(z, codebook, cluster_labels):
    bs = z.shape[0]
    idx = _nearest_idx(z, codebook)
    token_labels = jnp.take(cluster_labels, idx, axis=0)
    img = token_labels.reshape(bs, 1, 16, 16).astype(jnp.float32)
    out = jnp.repeat(jnp.repeat(img, 14, axis=2), 14, axis=3)
    return out
